# baseline (speedup 1.0000x reference)
import jax
import jax.numpy as jnp
from jax import lax
from jax.experimental import pallas as pl
from jax.experimental.pallas import tpu as pltpu
from jax.experimental.pallas import tpu_sc as plsc

N = 10000
E = 320000
D = 132
CW = 80
EH = 32

NC = 2
NS = 16
EPW = E // NS
K = 128
NCHUNK = 158
NPAIR = NCHUNK // 2
EPW_PAD = NCHUNK * K
NPAD = 10240
RPT = NPAD // NS
RB = 1000
GRID = N // RB


def _edge_net_body(ea, we1, be1, we2, be2, row_out):
    eh = jax.nn.relu(jnp.dot(ea[...], we1[...], preferred_element_type=jnp.float32)
                     + be1[...])
    row_out[...] = jnp.dot(eh, we2[...], preferred_element_type=jnp.float32) + be2[...]


def _proj_body(x, wp, bp, a_lo, a_hi, h_out, plo_out, phi_out):
    h = jax.nn.relu(jnp.dot(x[...], wp[...], preferred_element_type=jnp.float32)
                    + bp[...])
    h_out[...] = h
    plo_out[...] = lax.dot_general(h, a_lo[...], (((1,), (1,)), ((), ())),
                                   preferred_element_type=jnp.float32)
    phi_out[...] = lax.dot_general(h, a_hi[...], (((1,), (1,)), ((), ())),
                                   preferred_element_type=jnp.float32)


def _sc_scatter_body(zeros_hbm, srcp, dstp, plo, phi, out, srcv, dstv,
                     rows_a, rows_b, agg, sem_a, sem_b, sem_sb):
    c = lax.axis_index("c")
    s = lax.axis_index("s")
    pltpu.sync_copy(zeros_hbm, agg.at[pl.ds(s * RPT, RPT)])
    pltpu.sync_copy(srcp.at[s], srcv)
    pltpu.sync_copy(dstp.at[s], dstv)
    plsc.subcore_barrier()

    def pipeline(tab):
        pltpu.async_copy(tab.at[srcv.at[0]], rows_a, sem_a)

        def pair(t, carry):
            j0 = 2 * t
            j1 = j0 + 1

            @pl.when(t > 0)
            def _():
                pltpu.make_async_copy(rows_b, agg.at[dstv.at[j1 - 2]],
                                      sem_sb).wait()

            pltpu.async_copy(tab.at[srcv.at[j1]], rows_b, sem_b)
            pltpu.make_async_copy(tab.at[srcv.at[j0]], rows_a, sem_a).wait()
            pltpu.sync_copy(rows_a, agg.at[dstv.at[j0]], add=True)

            @pl.when(t + 1 < NPAIR)
            def _():
                pltpu.async_copy(tab.at[srcv.at[j0 + 2]], rows_a, sem_a)

            pltpu.make_async_copy(tab.at[srcv.at[j1]], rows_b, sem_b).wait()
            pltpu.async_copy(rows_b, agg.at[dstv.at[j1]], sem_sb, add=True)
            return carry

        lax.fori_loop(0, NPAIR, pair, 0)
        pltpu.make_async_copy(rows_b, agg.at[dstv.at[NCHUNK - 1]], sem_sb).wait()

    @pl.when(c == 0)
    def _():
        pipeline(plo)

    @pl.when(c == 1)
    def _():
        pipeline(phi)

    plsc.subcore_barrier()
    pltpu.sync_copy(agg.at[pl.ds(s * RPT, RPT)], out.at[c, pl.ds(s * RPT, RPT)])


def _gru_body(p0, p1, h, wr, wz, wn, vr, vz, vn, br, bz, bn, cr, cz, cn, out):
    hb = h[...]
    m = jax.nn.relu(jnp.concatenate([p0[0], p1[0]], axis=1)[:, :D])
    r = jax.nn.sigmoid(jnp.dot(m, wr[...], preferred_element_type=jnp.float32) + br[...]
                       + jnp.dot(hb, vr[...], preferred_element_type=jnp.float32) + cr[...])
    z = jax.nn.sigmoid(jnp.dot(m, wz[...], preferred_element_type=jnp.float32) + bz[...]
                       + jnp.dot(hb, vz[...], preferred_element_type=jnp.float32) + cz[...])
    n = jnp.tanh(jnp.dot(m, wn[...], preferred_element_type=jnp.float32) + bn[...]
                 + r * (jnp.dot(hb, vn[...], preferred_element_type=jnp.float32) + cn[...]))
    out[...] = (1.0 - z) * n + z * hb


def kernel(x, edge_index, edge_attr, W_proj, b_proj, We1, be1, We2, be2,
           W_ih, b_ih, W_hh, b_hh):
    f32 = jnp.float32

    arow = pl.pallas_call(
        _edge_net_body,
        out_shape=jax.ShapeDtypeStruct((1, D * D), f32),
    )(edge_attr[:1], We1, be1.reshape(1, EH), We2, be2.reshape(1, D * D))
    a_pad = jnp.pad(arow.reshape(D, D), ((0, 2 * CW - D), (0, 0)))
    a_lo, a_hi = a_pad[:CW], a_pad[CW:]

    row_spec = pl.BlockSpec((RB, D), lambda i: (i, 0))
    half_spec = pl.BlockSpec((RB, CW), lambda i: (i, 0))
    full = lambda shape: pl.BlockSpec(shape, lambda i: (0,) * len(shape))
    h_full, plo, phi = pl.pallas_call(
        _proj_body,
        grid=(GRID,),
        in_specs=[row_spec, full((D, D)), full((1, D)), full((CW, D)),
                  full((CW, D))],
        out_specs=[row_spec, half_spec, half_spec],
        out_shape=[jax.ShapeDtypeStruct((N, D), f32),
                   jax.ShapeDtypeStruct((NPAD, CW), f32),
                   jax.ShapeDtypeStruct((NPAD, CW), f32)],
    )(x, W_proj, b_proj.reshape(1, D), a_lo, a_hi)

    src = edge_index[0].reshape(NS, EPW)
    dst = edge_index[1].reshape(NS, EPW)
    pad = ((0, 0), (0, EPW_PAD - EPW))
    srcp = jnp.pad(src, pad).reshape(NS, NCHUNK, K)
    dstp = jnp.pad(dst, pad, constant_values=N).reshape(NS, NCHUNK, K)

    mesh = plsc.VectorSubcoreMesh(core_axis_name="c", subcore_axis_name="s")
    partials = pl.kernel(
        _sc_scatter_body,
        out_type=jax.ShapeDtypeStruct((NC, NPAD, CW), f32),
        mesh=mesh,
        scratch_types=[
            pltpu.VMEM((NCHUNK, K), jnp.int32),
            pltpu.VMEM((NCHUNK, K), jnp.int32),
            pltpu.VMEM((K, CW), f32),
            pltpu.VMEM((K, CW), f32),
            pltpu.VMEM_SHARED((NPAD, CW), f32),
            pltpu.SemaphoreType.DMA,
            pltpu.SemaphoreType.DMA,
            pltpu.SemaphoreType.DMA,
        ],
        compiler_params=pltpu.CompilerParams(use_tc_tiling_on_sc=False),
    )(jnp.zeros((RPT, CW), f32), srcp, dstp, plo, phi)

    wr, wz, wn = W_ih[:, :D], W_ih[:, D:2 * D], W_ih[:, 2 * D:]
    vr, vz, vn = W_hh[:, :D], W_hh[:, D:2 * D], W_hh[:, 2 * D:]
    br, bz, bn = (b_ih[:D].reshape(1, D), b_ih[D:2 * D].reshape(1, D),
                  b_ih[2 * D:].reshape(1, D))
    cr, cz, cn = (b_hh[:D].reshape(1, D), b_hh[D:2 * D].reshape(1, D),
                  b_hh[2 * D:].reshape(1, D))
    p_spec0 = pl.BlockSpec((1, RB, CW), lambda i: (0, i, 0))
    p_spec1 = pl.BlockSpec((1, RB, CW), lambda i: (1, i, 0))
    fd = full((D, D))
    fb = full((1, D))
    hidden = pl.pallas_call(
        _gru_body,
        grid=(GRID,),
        in_specs=[p_spec0, p_spec1, row_spec, fd, fd, fd, fd, fd, fd,
                  fb, fb, fb, fb, fb, fb],
        out_specs=row_spec,
        out_shape=jax.ShapeDtypeStruct((N, D), f32),
    )(partials, partials, h_full, wr, wz, wn, vr, vz, vn,
      br, bz, bn, cr, cz, cn)
    return hidden


# speedup vs baseline: 1.1901x; 1.1901x over previous
import jax
import jax.numpy as jnp
from jax import lax
from jax.experimental import pallas as pl
from jax.experimental.pallas import tpu as pltpu
from jax.experimental.pallas import tpu_sc as plsc

N = 10000
E = 320000
D = 132
CW = 80
EH = 32

NC = 2
NS = 16
EPW = E // NS
K = 80
NCHUNK = 250
NPAIR = NCHUNK // 2
NPAD = 10240
RPT = NPAD // NS
RB = 1000
GRID = N // RB


def _edge_net_body(ea, we1, be1, we2, be2, row_out):
    eh = jax.nn.relu(jnp.dot(ea[...], we1[...], preferred_element_type=jnp.float32)
                     + be1[...])
    row_out[...] = jnp.dot(eh, we2[...], preferred_element_type=jnp.float32) + be2[...]


def _proj_body(x, wp, bp, a_lo, a_hi, h_out, plo_out, phi_out):
    h = jax.nn.relu(jnp.dot(x[...], wp[...], preferred_element_type=jnp.float32)
                    + bp[...])
    h_out[...] = h
    plo_out[...] = lax.dot_general(h, a_lo[...], (((1,), (1,)), ((), ())),
                                   preferred_element_type=jnp.float32)
    phi_out[...] = lax.dot_general(h, a_hi[...], (((1,), (1,)), ((), ())),
                                   preferred_element_type=jnp.float32)


def _sc_scatter_body(zeros_hbm, srcp, dstp, plo, phi, out, srcv, dstv,
                     rows_a, rows_b, agg, sem_a, sem_b, sem_sb):
    c = lax.axis_index("c")
    s = lax.axis_index("s")
    pltpu.sync_copy(zeros_hbm, agg.at[pl.ds(s * RPT, RPT)])
    pltpu.sync_copy(srcp.at[s], srcv)
    pltpu.sync_copy(dstp.at[s], dstv)
    plsc.subcore_barrier()

    def pipeline(tab):
        pltpu.async_copy(tab.at[srcv.at[0]], rows_a, sem_a)

        def pair(t, carry):
            j0 = 2 * t
            j1 = j0 + 1

            @pl.when(t > 0)
            def _():
                pltpu.make_async_copy(rows_b, agg.at[dstv.at[j1 - 2]],
                                      sem_sb).wait()

            pltpu.async_copy(tab.at[srcv.at[j1]], rows_b, sem_b)
            pltpu.make_async_copy(tab.at[srcv.at[j0]], rows_a, sem_a).wait()
            pltpu.sync_copy(rows_a, agg.at[dstv.at[j0]], add=True)

            @pl.when(t + 1 < NPAIR)
            def _():
                pltpu.async_copy(tab.at[srcv.at[j0 + 2]], rows_a, sem_a)

            pltpu.make_async_copy(tab.at[srcv.at[j1]], rows_b, sem_b).wait()
            pltpu.async_copy(rows_b, agg.at[dstv.at[j1]], sem_sb, add=True)
            return carry

        lax.fori_loop(0, NPAIR, pair, 0)
        pltpu.make_async_copy(rows_b, agg.at[dstv.at[NCHUNK - 1]], sem_sb).wait()

    @pl.when(c == 0)
    def _():
        pipeline(plo)

    @pl.when(c == 1)
    def _():
        pipeline(phi)

    plsc.subcore_barrier()
    pltpu.sync_copy(agg.at[pl.ds(s * RPT, RPT)], out.at[c, pl.ds(s * RPT, RPT)])


def _gru_body(p0, p1, h, wr, wz, wn, vr, vz, vn, br, bz, bn, cr, cz, cn, out):
    hb = h[...]
    m = jax.nn.relu(jnp.concatenate([p0[0], p1[0]], axis=1)[:, :D])
    r = jax.nn.sigmoid(jnp.dot(m, wr[...], preferred_element_type=jnp.float32) + br[...]
                       + jnp.dot(hb, vr[...], preferred_element_type=jnp.float32) + cr[...])
    z = jax.nn.sigmoid(jnp.dot(m, wz[...], preferred_element_type=jnp.float32) + bz[...]
                       + jnp.dot(hb, vz[...], preferred_element_type=jnp.float32) + cz[...])
    n = jnp.tanh(jnp.dot(m, wn[...], preferred_element_type=jnp.float32) + bn[...]
                 + r * (jnp.dot(hb, vn[...], preferred_element_type=jnp.float32) + cn[...]))
    out[...] = (1.0 - z) * n + z * hb


def kernel(x, edge_index, edge_attr, W_proj, b_proj, We1, be1, We2, be2,
           W_ih, b_ih, W_hh, b_hh):
    f32 = jnp.float32

    arow = pl.pallas_call(
        _edge_net_body,
        out_shape=jax.ShapeDtypeStruct((1, D * D), f32),
    )(edge_attr[:1], We1, be1.reshape(1, EH), We2, be2.reshape(1, D * D))
    a_pad = jnp.pad(arow.reshape(D, D), ((0, 2 * CW - D), (0, 0)))
    a_lo, a_hi = a_pad[:CW], a_pad[CW:]

    row_spec = pl.BlockSpec((RB, D), lambda i: (i, 0))
    half_spec = pl.BlockSpec((RB, CW), lambda i: (i, 0))
    full = lambda shape: pl.BlockSpec(shape, lambda i: (0,) * len(shape))
    h_full, plo, phi = pl.pallas_call(
        _proj_body,
        grid=(GRID,),
        in_specs=[row_spec, full((D, D)), full((1, D)), full((CW, D)),
                  full((CW, D))],
        out_specs=[row_spec, half_spec, half_spec],
        out_shape=[jax.ShapeDtypeStruct((N, D), f32),
                   jax.ShapeDtypeStruct((NPAD, CW), f32),
                   jax.ShapeDtypeStruct((NPAD, CW), f32)],
    )(x, W_proj, b_proj.reshape(1, D), a_lo, a_hi)

    srcp = edge_index[0].reshape(NS, NCHUNK, K)
    dstp = edge_index[1].reshape(NS, NCHUNK, K)

    mesh = plsc.VectorSubcoreMesh(core_axis_name="c", subcore_axis_name="s")
    partials = pl.kernel(
        _sc_scatter_body,
        out_type=jax.ShapeDtypeStruct((NC, NPAD, CW), f32),
        mesh=mesh,
        scratch_types=[
            pltpu.VMEM((NCHUNK, K), jnp.int32),
            pltpu.VMEM((NCHUNK, K), jnp.int32),
            pltpu.VMEM((K, CW), f32),
            pltpu.VMEM((K, CW), f32),
            pltpu.VMEM_SHARED((NPAD, CW), f32),
            pltpu.SemaphoreType.DMA,
            pltpu.SemaphoreType.DMA,
            pltpu.SemaphoreType.DMA,
        ],
        compiler_params=pltpu.CompilerParams(use_tc_tiling_on_sc=False),
    )(jnp.zeros((RPT, CW), f32), srcp, dstp, plo, phi)

    wr, wz, wn = W_ih[:, :D], W_ih[:, D:2 * D], W_ih[:, 2 * D:]
    vr, vz, vn = W_hh[:, :D], W_hh[:, D:2 * D], W_hh[:, 2 * D:]
    br, bz, bn = (b_ih[:D].reshape(1, D), b_ih[D:2 * D].reshape(1, D),
                  b_ih[2 * D:].reshape(1, D))
    cr, cz, cn = (b_hh[:D].reshape(1, D), b_hh[D:2 * D].reshape(1, D),
                  b_hh[2 * D:].reshape(1, D))
    p_spec0 = pl.BlockSpec((1, RB, CW), lambda i: (0, i, 0))
    p_spec1 = pl.BlockSpec((1, RB, CW), lambda i: (1, i, 0))
    fd = full((D, D))
    fb = full((1, D))
    hidden = pl.pallas_call(
        _gru_body,
        grid=(GRID,),
        in_specs=[p_spec0, p_spec1, row_spec, fd, fd, fd, fd, fd, fd,
                  fb, fb, fb, fb, fb, fb],
        out_specs=row_spec,
        out_shape=jax.ShapeDtypeStruct((N, D), f32),
    )(partials, partials, h_full, wr, wz, wn, vr, vz, vn,
      br, bz, bn, cr, cz, cn)
    return hidden


# speedup vs baseline: 1.2381x; 1.0404x over previous
import jax
import jax.numpy as jnp
from jax import lax
from jax.experimental import pallas as pl
from jax.experimental.pallas import tpu as pltpu
from jax.experimental.pallas import tpu_sc as plsc

N = 10000
E = 320000
D = 132
CW = 80
EH = 32

NC = 2
NS = 16
EPW = E // NS
K = 80
NCHUNK = 250
NPAIR = NCHUNK // 2
NPAD = 10240
RPT = NPAD // NS
RB = 1000
GRID = N // RB


def _edge_net_body(ea, we1, be1, we2, be2, row_out):
    eh = jax.nn.relu(jnp.dot(ea[...], we1[...], preferred_element_type=jnp.float32)
                     + be1[...])
    row_out[...] = jnp.dot(eh, we2[...], preferred_element_type=jnp.float32) + be2[...]


def _proj_body(x, wp, bp, a_lo, a_hi, h_out, plo_out, phi_out):
    h = jax.nn.relu(jnp.dot(x[...], wp[...], preferred_element_type=jnp.float32)
                    + bp[...])
    h_out[...] = h
    plo_out[...] = lax.dot_general(h, a_lo[...], (((1,), (1,)), ((), ())),
                                   preferred_element_type=jnp.float32)
    phi_out[...] = lax.dot_general(h, a_hi[...], (((1,), (1,)), ((), ())),
                                   preferred_element_type=jnp.float32)


def _sc_scatter_body(zeros_hbm, srcp, dstp, plo, phi, out, srcv, dstv,
                     rows_a, rows_b, agg, sem_a, sem_b, sem_sb):
    c = lax.axis_index("c")
    s = lax.axis_index("s")
    pltpu.sync_copy(zeros_hbm, agg.at[pl.ds(s * RPT, RPT)])
    pltpu.sync_copy(srcp.at[s], srcv)
    pltpu.sync_copy(dstp.at[s], dstv)
    plsc.subcore_barrier()

    def pipeline(tab):
        pltpu.async_copy(tab.at[srcv.at[0]], rows_a, sem_a)

        def pair(t, carry):
            j0 = 2 * t
            j1 = j0 + 1

            @pl.when(t > 0)
            def _():
                pltpu.make_async_copy(rows_b, agg.at[dstv.at[j1 - 2]],
                                      sem_sb).wait()

            pltpu.async_copy(tab.at[srcv.at[j1]], rows_b, sem_b)
            pltpu.make_async_copy(tab.at[srcv.at[j0]], rows_a, sem_a).wait()
            pltpu.sync_copy(rows_a, agg.at[dstv.at[j0]], add=True)

            @pl.when(t + 1 < NPAIR)
            def _():
                pltpu.async_copy(tab.at[srcv.at[j0 + 2]], rows_a, sem_a)

            pltpu.make_async_copy(tab.at[srcv.at[j1]], rows_b, sem_b).wait()
            pltpu.async_copy(rows_b, agg.at[dstv.at[j1]], sem_sb, add=True)
            return carry

        lax.fori_loop(0, NPAIR, pair, 0)
        pltpu.make_async_copy(rows_b, agg.at[dstv.at[NCHUNK - 1]], sem_sb).wait()

    @pl.when(c == 0)
    def _():
        pipeline(plo)

    @pl.when(c == 1)
    def _():
        pipeline(phi)

    plsc.subcore_barrier()
    pltpu.sync_copy(agg.at[pl.ds(s * RPT, RPT)],
                    out.at[c, pl.ds(s * RPT, RPT), pl.ds(0, CW)])


def _gru_body(p0, p1, h, wr, wz, wn, vr, vz, vn, br, bz, bn, cr, cz, cn, out):
    hb = h[...]
    m = jax.nn.relu(jnp.concatenate([p0[0][:, :CW], p1[0][:, :CW]],
                                    axis=1)[:, :D])
    r = jax.nn.sigmoid(jnp.dot(m, wr[...], preferred_element_type=jnp.float32) + br[...]
                       + jnp.dot(hb, vr[...], preferred_element_type=jnp.float32) + cr[...])
    z = jax.nn.sigmoid(jnp.dot(m, wz[...], preferred_element_type=jnp.float32) + bz[...]
                       + jnp.dot(hb, vz[...], preferred_element_type=jnp.float32) + cz[...])
    n = jnp.tanh(jnp.dot(m, wn[...], preferred_element_type=jnp.float32) + bn[...]
                 + r * (jnp.dot(hb, vn[...], preferred_element_type=jnp.float32) + cn[...]))
    out[...] = (1.0 - z) * n + z * hb


def kernel(x, edge_index, edge_attr, W_proj, b_proj, We1, be1, We2, be2,
           W_ih, b_ih, W_hh, b_hh):
    f32 = jnp.float32

    arow = pl.pallas_call(
        _edge_net_body,
        out_shape=jax.ShapeDtypeStruct((1, D * D), f32),
    )(edge_attr[:1], We1, be1.reshape(1, EH), We2, be2.reshape(1, D * D))
    a_pad = jnp.pad(arow.reshape(D, D), ((0, 2 * CW - D), (0, 0)))
    a_lo, a_hi = a_pad[:CW], a_pad[CW:]

    row_spec = pl.BlockSpec((RB, D), lambda i: (i, 0))
    half_spec = pl.BlockSpec((RB, CW), lambda i: (i, 0))
    full = lambda shape: pl.BlockSpec(shape, lambda i: (0,) * len(shape))
    h_full, plo, phi = pl.pallas_call(
        _proj_body,
        grid=(GRID,),
        in_specs=[row_spec, full((D, D)), full((1, D)), full((CW, D)),
                  full((CW, D))],
        out_specs=[row_spec, half_spec, half_spec],
        out_shape=[jax.ShapeDtypeStruct((N, D), f32),
                   jax.ShapeDtypeStruct((NPAD, CW), f32),
                   jax.ShapeDtypeStruct((NPAD, CW), f32)],
    )(x, W_proj, b_proj.reshape(1, D), a_lo, a_hi)

    srcp = edge_index[0].reshape(NS, NCHUNK, K)
    dstp = edge_index[1].reshape(NS, NCHUNK, K)

    mesh = plsc.VectorSubcoreMesh(core_axis_name="c", subcore_axis_name="s")
    partials = pl.kernel(
        _sc_scatter_body,
        out_type=jax.ShapeDtypeStruct((NC, NPAD, 128), f32),
        mesh=mesh,
        scratch_types=[
            pltpu.VMEM((NCHUNK, K), jnp.int32),
            pltpu.VMEM((NCHUNK, K), jnp.int32),
            pltpu.VMEM((K, CW), f32),
            pltpu.VMEM((K, CW), f32),
            pltpu.VMEM_SHARED((NPAD, CW), f32),
            pltpu.SemaphoreType.DMA,
            pltpu.SemaphoreType.DMA,
            pltpu.SemaphoreType.DMA,
        ],
        compiler_params=pltpu.CompilerParams(use_tc_tiling_on_sc=False),
    )(jnp.zeros((RPT, CW), f32), srcp, dstp, plo, phi)

    wr, wz, wn = W_ih[:, :D], W_ih[:, D:2 * D], W_ih[:, 2 * D:]
    vr, vz, vn = W_hh[:, :D], W_hh[:, D:2 * D], W_hh[:, 2 * D:]
    br, bz, bn = (b_ih[:D].reshape(1, D), b_ih[D:2 * D].reshape(1, D),
                  b_ih[2 * D:].reshape(1, D))
    cr, cz, cn = (b_hh[:D].reshape(1, D), b_hh[D:2 * D].reshape(1, D),
                  b_hh[2 * D:].reshape(1, D))
    p_spec0 = pl.BlockSpec((1, RB, 128), lambda i: (0, i, 0))
    p_spec1 = pl.BlockSpec((1, RB, 128), lambda i: (1, i, 0))
    fd = full((D, D))
    fb = full((1, D))
    hidden = pl.pallas_call(
        _gru_body,
        grid=(GRID,),
        in_specs=[p_spec0, p_spec1, row_spec, fd, fd, fd, fd, fd, fd,
                  fb, fb, fb, fb, fb, fb],
        out_specs=row_spec,
        out_shape=jax.ShapeDtypeStruct((N, D), f32),
    )(partials, partials, h_full, wr, wz, wn, vr, vz, vn,
      br, bz, bn, cr, cz, cn)
    return hidden


# speedup vs baseline: 1.4820x; 1.1970x over previous
import jax
import jax.numpy as jnp
from jax import lax
from jax.experimental import pallas as pl
from jax.experimental.pallas import tpu as pltpu
from jax.experimental.pallas import tpu_sc as plsc

N = 10000
E = 320000
D = 132
CW = 80
EH = 32

NC = 2
NS = 16
K = 128
NCHT = E // K
CHT = NCHT // NS + 1
NPAD = 10240
RPT = NPAD // NS
RB = 1000
GRID = N // RB


def _edge_net_body(ea, we1, be1, we2, be2, row_out):
    eh = jax.nn.relu(jnp.dot(ea[...], we1[...], preferred_element_type=jnp.float32)
                     + be1[...])
    row_out[...] = jnp.dot(eh, we2[...], preferred_element_type=jnp.float32) + be2[...]


def _proj_body(x, wp, bp, a_lo, a_hi, h_out, plo_out, phi_out):
    h = jax.nn.relu(jnp.dot(x[...], wp[...], preferred_element_type=jnp.float32)
                    + bp[...])
    h_out[...] = h
    plo_out[...] = lax.dot_general(h, a_lo[...], (((1,), (1,)), ((), ())),
                                   preferred_element_type=jnp.float32)
    phi_out[...] = lax.dot_general(h, a_hi[...], (((1,), (1,)), ((), ())),
                                   preferred_element_type=jnp.float32)


def _sc_scatter_body(zeros_hbm, eic, plo, phi, out, eiv, rows_a, rows_b, agg,
                     sem_a, sem_b):
    c = lax.axis_index("c")
    s = lax.axis_index("s")
    start = (s * NCHT) // NS
    cnt = ((s + 1) * NCHT) // NS - start
    pltpu.sync_copy(zeros_hbm, agg.at[pl.ds(s * RPT, RPT)])
    pltpu.sync_copy(eic.at[pl.ds(start, CHT)], eiv)
    plsc.subcore_barrier()

    def pipeline(tab):
        pltpu.async_copy(tab.at[eiv.at[0, 0]], rows_a, sem_a)

        def step(t, carry):
            even = lax.rem(t, 2) == 0
            more = t + 1 < cnt

            @pl.when(jnp.logical_and(even, more))
            def _():
                pltpu.async_copy(tab.at[eiv.at[t + 1, 0]], rows_b, sem_b)

            @pl.when(jnp.logical_and(jnp.logical_not(even), more))
            def _():
                pltpu.async_copy(tab.at[eiv.at[t + 1, 0]], rows_a, sem_a)

            @pl.when(even)
            def _():
                pltpu.make_async_copy(tab.at[eiv.at[t, 0]], rows_a,
                                      sem_a).wait()
                pltpu.sync_copy(rows_a, agg.at[eiv.at[t, 1]], add=True)

            @pl.when(jnp.logical_not(even))
            def _():
                pltpu.make_async_copy(tab.at[eiv.at[t, 0]], rows_b,
                                      sem_b).wait()
                pltpu.sync_copy(rows_b, agg.at[eiv.at[t, 1]], add=True)

            return carry

        lax.fori_loop(0, cnt, step, 0)

    @pl.when(c == 0)
    def _():
        pipeline(plo)

    @pl.when(c == 1)
    def _():
        pipeline(phi)

    plsc.subcore_barrier()
    pltpu.sync_copy(agg.at[pl.ds(s * RPT, RPT)],
                    out.at[c, pl.ds(s * RPT, RPT), pl.ds(0, CW)])


def _gru_body(p0, p1, h, wr, wz, wn, vr, vz, vn, br, bz, bn, cr, cz, cn, out):
    hb = h[...]
    m = jax.nn.relu(jnp.concatenate([p0[0][:, :CW], p1[0][:, :CW]],
                                    axis=1)[:, :D])
    r = jax.nn.sigmoid(jnp.dot(m, wr[...], preferred_element_type=jnp.float32) + br[...]
                       + jnp.dot(hb, vr[...], preferred_element_type=jnp.float32) + cr[...])
    z = jax.nn.sigmoid(jnp.dot(m, wz[...], preferred_element_type=jnp.float32) + bz[...]
                       + jnp.dot(hb, vz[...], preferred_element_type=jnp.float32) + cz[...])
    n = jnp.tanh(jnp.dot(m, wn[...], preferred_element_type=jnp.float32) + bn[...]
                 + r * (jnp.dot(hb, vn[...], preferred_element_type=jnp.float32) + cn[...]))
    out[...] = (1.0 - z) * n + z * hb


def kernel(x, edge_index, edge_attr, W_proj, b_proj, We1, be1, We2, be2,
           W_ih, b_ih, W_hh, b_hh):
    f32 = jnp.float32

    arow = pl.pallas_call(
        _edge_net_body,
        out_shape=jax.ShapeDtypeStruct((1, D * D), f32),
    )(edge_attr[:1], We1, be1.reshape(1, EH), We2, be2.reshape(1, D * D))
    a_pad = jnp.pad(arow.reshape(D, D), ((0, 2 * CW - D), (0, 0)))
    a_lo, a_hi = a_pad[:CW], a_pad[CW:]

    row_spec = pl.BlockSpec((RB, D), lambda i: (i, 0))
    half_spec = pl.BlockSpec((RB, CW), lambda i: (i, 0))
    full = lambda shape: pl.BlockSpec(shape, lambda i: (0,) * len(shape))
    h_full, plo, phi = pl.pallas_call(
        _proj_body,
        grid=(GRID,),
        in_specs=[row_spec, full((D, D)), full((1, D)), full((CW, D)),
                  full((CW, D))],
        out_specs=[row_spec, half_spec, half_spec],
        out_shape=[jax.ShapeDtypeStruct((N, D), f32),
                   jax.ShapeDtypeStruct((NPAD, CW), f32),
                   jax.ShapeDtypeStruct((NPAD, CW), f32)],
    )(x, W_proj, b_proj.reshape(1, D), a_lo, a_hi)

    eic = edge_index.reshape(2, NCHT, K).transpose(1, 0, 2)

    mesh = plsc.VectorSubcoreMesh(core_axis_name="c", subcore_axis_name="s")
    partials = pl.kernel(
        _sc_scatter_body,
        out_type=jax.ShapeDtypeStruct((NC, NPAD, 128), f32),
        mesh=mesh,
        scratch_types=[
            pltpu.VMEM((CHT, 2, K), jnp.int32),
            pltpu.VMEM((K, CW), f32),
            pltpu.VMEM((K, CW), f32),
            pltpu.VMEM_SHARED((NPAD, CW), f32),
            pltpu.SemaphoreType.DMA,
            pltpu.SemaphoreType.DMA,
        ],
        compiler_params=pltpu.CompilerParams(use_tc_tiling_on_sc=False),
    )(jnp.zeros((RPT, CW), f32), eic, plo, phi)

    wr, wz, wn = W_ih[:, :D], W_ih[:, D:2 * D], W_ih[:, 2 * D:]
    vr, vz, vn = W_hh[:, :D], W_hh[:, D:2 * D], W_hh[:, 2 * D:]
    br, bz, bn = (b_ih[:D].reshape(1, D), b_ih[D:2 * D].reshape(1, D),
                  b_ih[2 * D:].reshape(1, D))
    cr, cz, cn = (b_hh[:D].reshape(1, D), b_hh[D:2 * D].reshape(1, D),
                  b_hh[2 * D:].reshape(1, D))
    p_spec0 = pl.BlockSpec((1, RB, 128), lambda i: (0, i, 0))
    p_spec1 = pl.BlockSpec((1, RB, 128), lambda i: (1, i, 0))
    fd = full((D, D))
    fb = full((1, D))
    hidden = pl.pallas_call(
        _gru_body,
        grid=(GRID,),
        in_specs=[p_spec0, p_spec1, row_spec, fd, fd, fd, fd, fd, fd,
                  fb, fb, fb, fb, fb, fb],
        out_specs=row_spec,
        out_shape=jax.ShapeDtypeStruct((N, D), f32),
    )(partials, partials, h_full, wr, wz, wn, vr, vz, vn,
      br, bz, bn, cr, cz, cn)
    return hidden


# speedup vs baseline: 1.4882x; 1.0042x over previous
import jax
import jax.numpy as jnp
from jax import lax
from jax.experimental import pallas as pl
from jax.experimental.pallas import tpu as pltpu
from jax.experimental.pallas import tpu_sc as plsc

N = 10000
E = 320000
D = 132
CW = 80
EH = 32

NC = 2
NS = 16
K = 128
NCHT = E // K
CHT = NCHT // NS + 1
NPAD = 10240
RPT = NPAD // NS
RB = 1000
GRID = N // RB


def _edge_net_body(ea, we1, be1, we2, be2, row_out):
    eh = jax.nn.relu(jnp.dot(ea[...], we1[...], preferred_element_type=jnp.float32)
                     + be1[...])
    row_out[...] = jnp.dot(eh, we2[...], preferred_element_type=jnp.float32) + be2[...]


def _proj_body(x, wp, bp, a_lo, a_hi, plo_out, phi_out):
    h = jax.nn.relu(jnp.dot(x[...], wp[...], preferred_element_type=jnp.float32)
                    + bp[...])
    plo_out[...] = lax.dot_general(h, a_lo[...], (((1,), (1,)), ((), ())),
                                   preferred_element_type=jnp.float32)
    phi_out[...] = lax.dot_general(h, a_hi[...], (((1,), (1,)), ((), ())),
                                   preferred_element_type=jnp.float32)


def _sc_scatter_body(zeros_hbm, eic, plo, phi, out, eiv, rows_a, rows_b, agg,
                     sem_a, sem_b):
    c = lax.axis_index("c")
    s = lax.axis_index("s")
    start = (s * NCHT) // NS
    cnt = ((s + 1) * NCHT) // NS - start
    pltpu.sync_copy(zeros_hbm, agg.at[pl.ds(s * RPT, RPT)])
    pltpu.sync_copy(eic.at[pl.ds(start, CHT)], eiv)
    plsc.subcore_barrier()

    def pipeline(tab):
        pltpu.async_copy(tab.at[eiv.at[0, 0]], rows_a, sem_a)

        def step(t, carry):
            even = lax.rem(t, 2) == 0
            more = t + 1 < cnt

            @pl.when(jnp.logical_and(even, more))
            def _():
                pltpu.async_copy(tab.at[eiv.at[t + 1, 0]], rows_b, sem_b)

            @pl.when(jnp.logical_and(jnp.logical_not(even), more))
            def _():
                pltpu.async_copy(tab.at[eiv.at[t + 1, 0]], rows_a, sem_a)

            @pl.when(even)
            def _():
                pltpu.make_async_copy(tab.at[eiv.at[t, 0]], rows_a,
                                      sem_a).wait()
                pltpu.sync_copy(rows_a, agg.at[eiv.at[t, 1]], add=True)

            @pl.when(jnp.logical_not(even))
            def _():
                pltpu.make_async_copy(tab.at[eiv.at[t, 0]], rows_b,
                                      sem_b).wait()
                pltpu.sync_copy(rows_b, agg.at[eiv.at[t, 1]], add=True)

            return carry

        lax.fori_loop(0, cnt, step, 0)

    @pl.when(c == 0)
    def _():
        pipeline(plo)

    @pl.when(c == 1)
    def _():
        pipeline(phi)

    plsc.subcore_barrier()
    pltpu.sync_copy(agg.at[pl.ds(s * RPT, RPT)],
                    out.at[c, pl.ds(s * RPT, RPT), pl.ds(0, CW)])


def _gru_body(p0, p1, x, wp, bp, wr, wz, wn, vr, vz, vn, br, bz, bn, cr, cz,
              cn, out):
    hb = jax.nn.relu(jnp.dot(x[...], wp[...], preferred_element_type=jnp.float32)
                     + bp[...])
    m = jax.nn.relu(jnp.concatenate([p0[0][:, :CW], p1[0][:, :CW]],
                                    axis=1)[:, :D])
    r = jax.nn.sigmoid(jnp.dot(m, wr[...], preferred_element_type=jnp.float32) + br[...]
                       + jnp.dot(hb, vr[...], preferred_element_type=jnp.float32) + cr[...])
    z = jax.nn.sigmoid(jnp.dot(m, wz[...], preferred_element_type=jnp.float32) + bz[...]
                       + jnp.dot(hb, vz[...], preferred_element_type=jnp.float32) + cz[...])
    n = jnp.tanh(jnp.dot(m, wn[...], preferred_element_type=jnp.float32) + bn[...]
                 + r * (jnp.dot(hb, vn[...], preferred_element_type=jnp.float32) + cn[...]))
    out[...] = (1.0 - z) * n + z * hb


def kernel(x, edge_index, edge_attr, W_proj, b_proj, We1, be1, We2, be2,
           W_ih, b_ih, W_hh, b_hh):
    f32 = jnp.float32

    arow = pl.pallas_call(
        _edge_net_body,
        out_shape=jax.ShapeDtypeStruct((1, D * D), f32),
    )(edge_attr[:1], We1, be1.reshape(1, EH), We2, be2.reshape(1, D * D))
    a_pad = jnp.pad(arow.reshape(D, D), ((0, 2 * CW - D), (0, 0)))
    a_lo, a_hi = a_pad[:CW], a_pad[CW:]

    row_spec = pl.BlockSpec((RB, D), lambda i: (i, 0))
    half_spec = pl.BlockSpec((RB, CW), lambda i: (i, 0))
    full = lambda shape: pl.BlockSpec(shape, lambda i: (0,) * len(shape))
    plo, phi = pl.pallas_call(
        _proj_body,
        grid=(GRID,),
        in_specs=[row_spec, full((D, D)), full((1, D)), full((CW, D)),
                  full((CW, D))],
        out_specs=[half_spec, half_spec],
        out_shape=[jax.ShapeDtypeStruct((NPAD, CW), f32),
                   jax.ShapeDtypeStruct((NPAD, CW), f32)],
    )(x, W_proj, b_proj.reshape(1, D), a_lo, a_hi)

    eic = edge_index.reshape(2, NCHT, K).transpose(1, 0, 2)

    mesh = plsc.VectorSubcoreMesh(core_axis_name="c", subcore_axis_name="s")
    partials = pl.kernel(
        _sc_scatter_body,
        out_type=jax.ShapeDtypeStruct((NC, NPAD, 128), f32),
        mesh=mesh,
        scratch_types=[
            pltpu.VMEM((CHT, 2, K), jnp.int32),
            pltpu.VMEM((K, CW), f32),
            pltpu.VMEM((K, CW), f32),
            pltpu.VMEM_SHARED((NPAD, CW), f32),
            pltpu.SemaphoreType.DMA,
            pltpu.SemaphoreType.DMA,
        ],
        compiler_params=pltpu.CompilerParams(use_tc_tiling_on_sc=False),
    )(jnp.zeros((RPT, CW), f32), eic, plo, phi)

    wr, wz, wn = W_ih[:, :D], W_ih[:, D:2 * D], W_ih[:, 2 * D:]
    vr, vz, vn = W_hh[:, :D], W_hh[:, D:2 * D], W_hh[:, 2 * D:]
    br, bz, bn = (b_ih[:D].reshape(1, D), b_ih[D:2 * D].reshape(1, D),
                  b_ih[2 * D:].reshape(1, D))
    cr, cz, cn = (b_hh[:D].reshape(1, D), b_hh[D:2 * D].reshape(1, D),
                  b_hh[2 * D:].reshape(1, D))
    p_spec0 = pl.BlockSpec((1, RB, 128), lambda i: (0, i, 0))
    p_spec1 = pl.BlockSpec((1, RB, 128), lambda i: (1, i, 0))
    fd = full((D, D))
    fb = full((1, D))
    hidden = pl.pallas_call(
        _gru_body,
        grid=(GRID,),
        in_specs=[p_spec0, p_spec1, row_spec, fd, fb, fd, fd, fd, fd, fd, fd,
                  fb, fb, fb, fb, fb, fb],
        out_specs=row_spec,
        out_shape=jax.ShapeDtypeStruct((N, D), f32),
    )(partials, partials, x, W_proj, b_proj.reshape(1, D), wr, wz, wn,
      vr, vz, vn, br, bz, bn, cr, cz, cn)
    return hidden
